# Initial kernel scaffold; baseline (speedup 1.0000x reference)
import jax
import jax.numpy as jnp
from jax.experimental import pallas as pl


def kernel(point_rates, spatial, gia, all_source_coords):
    raise NotImplementedError("write your pallas kernel here")


# speedup vs baseline: 22.4634x; 22.4634x over previous
import jax
import jax.numpy as jnp
from jax.experimental import pallas as pl


def _body(coords_ref, pr_ref, spatial_ref, gia_ref, out_ref):
    H, W = out_ref.shape[2], out_ref.shape[3]
    S = coords_ref.shape[2]
    s = spatial_ref[0]
    xmin = jnp.min(s[0])
    xmax = jnp.max(s[0])
    ymin = jnp.min(s[1])
    ymax = jnp.max(s[1])
    c = coords_ref[0]
    cx = c[0:1, :]
    cy = c[1:2, :]
    nx = (cx - xmin) / (xmax - xmin)
    ny = (cy - ymin) / (ymax - ymin)
    fx = jnp.clip(jnp.round(nx * (W - 1)), 0.0, W - 1)
    fy = jnp.clip(jnp.round(ny * (H - 1)), 0.0, H - 1)
    px = fx.astype(jnp.int32)
    py = fy.astype(jnp.int32)
    in_view = ((cx >= xmin) & (cx <= xmax) & (cy >= ymin) & (cy <= ymax))
    ih = jax.lax.broadcasted_iota(jnp.int32, (H, S), 0)
    iw = jax.lax.broadcasted_iota(jnp.int32, (W, S), 0)
    oh_y = (ih == py).astype(jnp.float32)
    oh_x = (iw == px).astype(jnp.float32)
    gia = gia_ref[0]
    rows = jax.lax.dot_general(gia, oh_y, (((0,), (0,)), ((), ())),
                               preferred_element_type=jnp.float32)
    g = jnp.sum(rows * oh_x, axis=0, keepdims=True)
    e = pr_ref[0] * g * in_view.astype(jnp.float32)
    a = oh_y * e
    field = jax.lax.dot_general(a, oh_x, (((1,), (1,)), ((), ())),
                                preferred_element_type=jnp.float32)
    out_ref[0, 0] = field


def kernel(point_rates, spatial, gia, all_source_coords):
    B, H, W = gia.shape
    S = all_source_coords.shape[0]
    coords3 = jnp.transpose(all_source_coords)[None]
    pr3 = point_rates[:, None, :]
    out = pl.pallas_call(
        _body,
        grid=(B,),
        in_specs=[
            pl.BlockSpec((1, 2, S), lambda b: (0, 0, 0)),
            pl.BlockSpec((1, 1, S), lambda b: (b, 0, 0)),
            pl.BlockSpec((1, 2, H, W), lambda b: (b, 0, 0, 0)),
            pl.BlockSpec((1, H, W), lambda b: (b, 0, 0)),
        ],
        out_specs=pl.BlockSpec((1, 1, H, W), lambda b: (b, 0, 0, 0)),
        out_shape=jax.ShapeDtypeStruct((B, 1, H, W), jnp.float32),
    )(coords3, pr3, spatial, gia)
    return out
